# Initial kernel scaffold; baseline (speedup 1.0000x reference)
import jax
import jax.numpy as jnp
from jax.experimental import pallas as pl


def kernel(seq, token_table, pos_table):
    raise NotImplementedError("write your pallas kernel here")


# speedup vs baseline: 1.4251x; 1.4251x over previous
import functools

import jax
import jax.numpy as jnp
from jax import lax
from jax.experimental import pallas as pl
from jax.experimental.pallas import tpu as pltpu
from jax.experimental.pallas import tpu_sc as plsc

B = 4096
L = 200
D = 32
LANES = 16
NC = 2
NS = 16
NW = NC * NS

LOOK = B * L
W_IDX = 100
ROWS_TOTAL = LOOK // W_IDX
PER_W = LOOK // NW
CHUNK = 1600
G = CHUNK // W_IDX
NCHUNK = PER_W // CHUNK
PERIODS = CHUNK // L


def _body(seq_hbm, tok_hbm, pos_hbm, out_hbm, idx_v, rows_v, pos_v, gsem):
    wid = lax.axis_index("s") * NC + lax.axis_index("c")
    pltpu.sync_copy(pos_hbm, pos_v)
    row0 = wid * (PER_W // W_IDX)

    def chunk_body(c, carry):
        pltpu.sync_copy(seq_hbm.at[pl.ds(row0 + c * G, G)], idx_v)
        copies = [
            pltpu.async_copy(
                tok_hbm.at[idx_v.at[g]],
                rows_v.at[pl.ds(g * W_IDX, W_IDX)],
                gsem,
            )
            for g in range(G)
        ]
        for cp in copies:
            cp.wait()

        def add_body(j, inner):
            pa = pos_v[j, pl.ds(0, LANES)]
            pb = pos_v[j, pl.ds(LANES, LANES)]
            for r in range(PERIODS):
                row = r * L + j
                rows_v[row, pl.ds(0, LANES)] += pa
                rows_v[row, pl.ds(LANES, LANES)] += pb
            return inner

        lax.fori_loop(0, L, add_body, 0)
        out_base = wid * PER_W + c * CHUNK
        pltpu.sync_copy(rows_v, out_hbm.at[pl.ds(out_base, CHUNK)])
        return carry

    lax.fori_loop(0, NCHUNK, chunk_body, 0)


def kernel(seq, token_table, pos_table):
    seq2 = seq.reshape(ROWS_TOTAL, W_IDX)
    mesh = plsc.VectorSubcoreMesh(core_axis_name="c", subcore_axis_name="s")
    call = pl.kernel(
        _body,
        out_type=jax.ShapeDtypeStruct((LOOK, D), jnp.float32),
        mesh=mesh,
        compiler_params=pltpu.CompilerParams(use_tc_tiling_on_sc=False),
        scratch_types=[
            pltpu.VMEM((G, W_IDX), jnp.int32),
            pltpu.VMEM((CHUNK, D), jnp.float32),
            pltpu.VMEM((L, D), jnp.float32),
            pltpu.SemaphoreType.DMA,
        ],
    )
    out = call(seq2, token_table, pos_table)
    return out.reshape(B, L, D)


# speedup vs baseline: 1.4281x; 1.0021x over previous
import functools

import jax
import jax.numpy as jnp
from jax import lax
from jax.experimental import pallas as pl
from jax.experimental.pallas import tpu as pltpu
from jax.experimental.pallas import tpu_sc as plsc

B = 4096
L = 200
D = 32
LANES = 16
NC = 2
NS = 16
NW = NC * NS

PER_W_B = B // NW
CHUNK_B = 8
NCHUNK = PER_W_B // CHUNK_B
SPLITS = ((0, 104), (104, 96))


def _body(seq_hbm, tok_hbm, pos_hbm, out_hbm, idx_v, rows_v, pos_v, gsem):
    wid = lax.axis_index("s") * NC + lax.axis_index("c")
    pltpu.sync_copy(pos_hbm, pos_v)
    b0 = wid * PER_W_B

    def chunk_body(c, carry):
        base = b0 + c * CHUNK_B
        pltpu.sync_copy(seq_hbm.at[pl.ds(base, CHUNK_B)], idx_v)
        copies = []
        for b in range(CHUNK_B):
            for off, width in SPLITS:
                copies.append(pltpu.async_copy(
                    tok_hbm.at[idx_v.at[b, pl.ds(off, width)]],
                    rows_v.at[b, pl.ds(off, width)],
                    gsem,
                ))
        for cp in copies:
            cp.wait()

        def add_body(j, inner):
            pa = pos_v[j, pl.ds(0, LANES)]
            pb = pos_v[j, pl.ds(LANES, LANES)]
            for b in range(CHUNK_B):
                rows_v[b, j, pl.ds(0, LANES)] += pa
                rows_v[b, j, pl.ds(LANES, LANES)] += pb
            return inner

        lax.fori_loop(0, L, add_body, 0)
        pltpu.sync_copy(rows_v, out_hbm.at[pl.ds(base, CHUNK_B)])
        return carry

    lax.fori_loop(0, NCHUNK, chunk_body, 0)


def kernel(seq, token_table, pos_table):
    mesh = plsc.VectorSubcoreMesh(core_axis_name="c", subcore_axis_name="s")
    call = pl.kernel(
        _body,
        out_type=jax.ShapeDtypeStruct((B, L, D), jnp.float32),
        mesh=mesh,
        compiler_params=pltpu.CompilerParams(use_tc_tiling_on_sc=False),
        scratch_types=[
            pltpu.VMEM((CHUNK_B, L), jnp.int32),
            pltpu.VMEM((CHUNK_B, L, D), jnp.float32),
            pltpu.VMEM((L, D), jnp.float32),
            pltpu.SemaphoreType.DMA,
        ],
    )
    return call(seq, token_table, pos_table)
